# baseline (speedup 1.0000x reference)
import dataclasses
import functools

import jax
import jax.numpy as jnp
from jax import lax
from jax.experimental import pallas as pl
from jax.experimental.pallas import tpu as pltpu
from jax.experimental.pallas import tpu_sc as plsc

_SAFE1 = 0.1
_SAFE2 = 0.5

_ROWS = 8192
_D = 128
_NCENT = 1000
_NW = 32
_RPW = _ROWS // _NW
_LANES = 16
_BLK = 1024


def _sc_gather(labels2, centroids, delta):
  mesh = plsc.VectorSubcoreMesh(core_axis_name="c", subcore_axis_name="s")
  f32 = jnp.float32
  cp = pltpu.CompilerParams()
  if "needs_layout_passes" in pltpu.CompilerParams.__dataclass_fields__:
    cp = dataclasses.replace(cp, needs_layout_passes=False)

  @functools.partial(
      pl.kernel,
      compiler_params=cp,
      out_type=(
          jax.ShapeDtypeStruct((_ROWS, _D), f32),
          jax.ShapeDtypeStruct((_ROWS,), f32),
      ),
      mesh=mesh,
      scratch_types=[
          pltpu.VMEM((2, 128), jnp.int32),
          pltpu.VMEM((_RPW, _D), f32),
          pltpu.VMEM((_NCENT,), f32),
          pltpu.VMEM((_RPW,), f32),
          pltpu.SemaphoreType.DMA,
          pltpu.SemaphoreType.DMA,
          pltpu.SemaphoreType.DMA,
      ],
  )
  def sc_kernel(labels_hbm, cent_hbm, delta_hbm,
                cg_hbm, dg_hbm,
                lbl_v, c_v, dtab_v, dg_v,
                sem_a, sem_c, sem_g):
    wid = lax.axis_index("s") * 2 + lax.axis_index("c")
    base = wid * _RPW

    cp_lbl = pltpu.async_copy(labels_hbm.at[pl.ds(wid * 2, 2)], lbl_v, sem_a)
    cp_dt = pltpu.async_copy(delta_hbm, dtab_v, sem_c)
    cp_lbl.wait()

    cp_g0 = pltpu.async_copy(
        cent_hbm.at[lbl_v.at[0]], c_v.at[pl.ds(0, 128)], sem_g)
    cp_g1 = pltpu.async_copy(
        cent_hbm.at[lbl_v.at[1]], c_v.at[pl.ds(128, 128)], sem_g)

    cp_dt.wait()
    for t in range(_RPW // _LANES):
      idx = lbl_v[t // 8, pl.ds((t % 8) * _LANES, _LANES)]
      dg_v[pl.ds(t * _LANES, _LANES)] = plsc.load_gather(dtab_v, [idx])
    o2 = pltpu.async_copy(dg_v, dg_hbm.at[pl.ds(base, _RPW)], sem_c)

    cp_g0.wait()
    cp_g1.wait()
    o1 = pltpu.async_copy(c_v, cg_hbm.at[pl.ds(base, _RPW)], sem_g)
    o1.wait()
    o2.wait()

  return sc_kernel(labels2, centroids, delta)


def _tc_loss(pooled3, cg, dg, delta):
  f32 = jnp.float32
  nblk = _ROWS // _BLK

  def body(po_ref, cg_ref, dg_ref, delta_ref, loss_ref, dsp_ref):
    b = pl.program_id(0)
    x = po_ref[:, 0, :]
    n = po_ref[:, 1, :]
    c = cg_ref[...]
    dx = x - c
    dn = n - c
    sx = jnp.sum(dx * dx, axis=1)
    sn = jnp.sum(dn * dn, axis=1)
    euc = jnp.sqrt(sx)
    neu = jnp.sqrt(sn)
    d = jax.nn.softplus(dg_ref[pl.ds(b * _BLK, _BLK)])
    pos = jnp.maximum(euc - d, 0.0)
    neg = jnp.maximum(d - euc, 0.0)
    npos = jnp.maximum(neu - (d + _SAFE2), 0.0)
    nneg = jnp.maximum((d - neu) + _SAFE1, 0.0)
    partial = (jnp.sum(pos) + jnp.sum(neg)) + (jnp.sum(npos) + jnp.sum(nneg))

    @pl.when(b == 0)
    def _():
      loss_ref[...] = jnp.zeros((1, 1), f32)
      dsp_ref[...] = jax.nn.softplus(delta_ref[...])

    loss_ref[...] += jnp.broadcast_to(partial * (1.0 / _ROWS), (1, 1))

  return pl.pallas_call(
      body,
      grid=(nblk,),
      in_specs=[
          pl.BlockSpec((_BLK, 2, _D), lambda b: (b, 0, 0)),
          pl.BlockSpec((_BLK, _D), lambda b: (b, 0)),
          pl.BlockSpec((_ROWS,), lambda b: (0,)),
          pl.BlockSpec((_NCENT,), lambda b: (0,)),
      ],
      out_specs=(
          pl.BlockSpec((1, 1), lambda b: (0, 0)),
          pl.BlockSpec((_NCENT,), lambda b: (0,)),
      ),
      out_shape=(
          jax.ShapeDtypeStruct((1, 1), f32),
          jax.ShapeDtypeStruct((_NCENT,), f32),
      ),
  )(pooled3, cg, dg, delta)


def kernel(pooled_output, centroids, labels, delta):
  pooled3 = pooled_output.reshape(_ROWS, 2, _D)
  labels2 = labels.reshape(_ROWS // 128, 128)
  cg, dg = _sc_gather(labels2, centroids, delta)
  loss2, dsp = _tc_loss(pooled3, cg, dg, delta)
  return loss2[0, 0], dsp
